# baseline (speedup 1.0000x reference)
import functools

import jax
import jax.numpy as jnp
from jax import lax
from jax.experimental import pallas as pl
from jax.experimental.pallas import tpu as pltpu
from jax.experimental.pallas import tpu_sc as plsc

_N = 10000
_E = 320000
_D = 128

_NC = 2
_NS = 16
_NW = _NC * _NS
_EPW = _E // _NW
_CH = 80
_NCH = _EPW // _CH
_HCH = 32
_NP = 10112
_RPT = _NP // _NS

@functools.cache
def _make_sc_agg():
    mesh = plsc.VectorSubcoreMesh(core_axis_name="c", subcore_axis_name="s")

    @functools.partial(
        pl.kernel,
        out_type=jax.ShapeDtypeStruct((_NC * _NP, _D), jnp.float32),
        mesh=mesh,
        scratch_types=[
            pltpu.VMEM((_HCH, _CH), jnp.int32),
            pltpu.VMEM((_HCH, _CH), jnp.int32),
            pltpu.VMEM((_CH, _D), jnp.float32),
            pltpu.VMEM((_CH, _D), jnp.float32),
            pltpu.VMEM((_CH, _D), jnp.float32),
            pltpu.VMEM((_CH, _D), jnp.float32),
            pltpu.VMEM_SHARED((_NP, _D), jnp.float32),
            pltpu.SemaphoreType.DMA,
            pltpu.SemaphoreType.DMA,
            pltpu.SemaphoreType.DMA,
            pltpu.SemaphoreType.DMA,
        ],
    )
    def sc_agg(x_hbm, src_hbm, dst_hbm, zeros_hbm, out_hbm,
               src_v, dst_v, rows_a, rows_b, rows_c, rows_d, agg_sh,
               sem_a, sem_b, sem_c, sem_d):
        cid = lax.axis_index("c")
        sid = lax.axis_index("s")
        wid = cid * _NS + sid
        row0 = sid * _RPT
        pltpu.sync_copy(zeros_hbm.at[pl.ds(row0, _RPT)],
                        agg_sh.at[pl.ds(row0, _RPT)])
        plsc.subcore_barrier()

        def gather(j, rv, sem):
            pltpu.async_copy(x_hbm.at[src_v.at[j]], rv, sem)

        def drain_scatter(j, rv, sem):
            pltpu.make_async_copy(x_hbm.at[src_v.at[j]], rv, sem).wait()
            pltpu.sync_copy(rv, agg_sh.at[dst_v.at[j]], add=True)

        def half(base, n):
            pltpu.sync_copy(src_hbm.at[wid, pl.ds(base, n)],
                            src_v.at[pl.ds(0, n)])
            pltpu.sync_copy(dst_hbm.at[wid, pl.ds(base, n)],
                            dst_v.at[pl.ds(0, n)])
            gather(0, rows_a, sem_a)
            gather(1, rows_b, sem_b)
            gather(2, rows_c, sem_c)
            gather(3, rows_d, sem_d)
            ntr = (n - 4) // 4

            def body(i, carry):
                ja = 4 * i
                drain_scatter(ja, rows_a, sem_a)
                gather(ja + 4, rows_a, sem_a)
                drain_scatter(ja + 1, rows_b, sem_b)
                gather(ja + 5, rows_b, sem_b)
                drain_scatter(ja + 2, rows_c, sem_c)
                gather(ja + 6, rows_c, sem_c)
                drain_scatter(ja + 3, rows_d, sem_d)
                gather(ja + 7, rows_d, sem_d)
                return carry

            lax.fori_loop(0, ntr, body, 0)
            j0 = 4 * ntr
            if n - j0 == 4:
                drain_scatter(j0, rows_a, sem_a)
                drain_scatter(j0 + 1, rows_b, sem_b)
                drain_scatter(j0 + 2, rows_c, sem_c)
                drain_scatter(j0 + 3, rows_d, sem_d)
            else:
                drain_scatter(j0, rows_a, sem_a)
                gather(j0 + 4, rows_a, sem_a)
                drain_scatter(j0 + 1, rows_b, sem_b)
                drain_scatter(j0 + 2, rows_c, sem_c)
                drain_scatter(j0 + 3, rows_d, sem_d)
                drain_scatter(j0 + 4, rows_a, sem_a)

        half(0, _HCH)
        half(_HCH, _HCH)
        half(2 * _HCH, _HCH)
        half(3 * _HCH, _NCH - 3 * _HCH)
        plsc.subcore_barrier()
        pltpu.sync_copy(agg_sh.at[pl.ds(row0, _RPT)],
                        out_hbm.at[pl.ds(cid * _NP + row0, _RPT)])

    return sc_agg


def _sc_agg(x, src3, dst3, zeros):
    return _make_sc_agg()(x, src3, dst3, zeros)


_BLK = 2000
_GRID = _N // _BLK


def _phase0(i, x_ref, a0_ref, a1_ref, w1_ref, b1_ref, hpre_ref, st_ref):
    s = x_ref[...] + a0_ref[...] + a1_ref[...]
    h = (jnp.dot(s, w1_ref[...], preferred_element_type=jnp.float32)
         + b1_ref[...])
    hpre_ref[pl.ds(i * _BLK, _BLK), :] = h

    @pl.when(i == 0)
    def _():
        st_ref[...] = jnp.zeros_like(st_ref)

    su = jnp.sum(h, axis=0, keepdims=True)
    sq = jnp.sum(h * h, axis=0, keepdims=True)
    st_ref[...] += jnp.concatenate([su, sq], axis=0)


def _normalize(i, g_ref, bt_ref, hpre_ref, st_ref):
    mean = st_ref[0:1, :] * (1.0 / _N)
    var = st_ref[1:2, :] * (1.0 / _N) - mean * mean
    scale = g_ref[...] * lax.rsqrt(var + 128.0)
    h = hpre_ref[pl.ds(i * _BLK, _BLK), :]
    return jnp.maximum((h - mean) * scale + bt_ref[...], 0.0)


def _conv_body(x_ref, a0_ref, a1_ref, w1_ref, b1_ref, g_ref, bt_ref,
               w2_ref, b2_ref, o_ref, hpre_ref, st_ref):
    p = pl.program_id(0)
    i = pl.program_id(1)

    @pl.when(p == 0)
    def _():
        _phase0(i, x_ref, a0_ref, a1_ref, w1_ref, b1_ref, hpre_ref, st_ref)

    @pl.when(p == 1)
    def _():
        hn = _normalize(i, g_ref, bt_ref, hpre_ref, st_ref)
        o_ref[...] = jnp.maximum(
            jnp.dot(hn, w2_ref[...], preferred_element_type=jnp.float32)
            + b2_ref[...], 0.0)


def _blk_map(p, i):
    return (i * (1 - p), 0)


def _blk_map2(p, i):
    return (i, 0)


def _full_map(p, i):
    return (0, 0)


_conv_tc = pl.pallas_call(
    _conv_body,
    grid=(2, _GRID),
    in_specs=[
        pl.BlockSpec((_BLK, _D), _blk_map),
        pl.BlockSpec((_BLK, _D), _blk_map),
        pl.BlockSpec((_BLK, _D), _blk_map),
        pl.BlockSpec((_D, _D), _full_map),
        pl.BlockSpec((1, _D), _full_map),
        pl.BlockSpec((1, _D), _full_map),
        pl.BlockSpec((1, _D), _full_map),
        pl.BlockSpec((_D, _D), _full_map),
        pl.BlockSpec((1, _D), _full_map),
    ],
    out_specs=pl.BlockSpec((_BLK, _D), _blk_map2),
    out_shape=jax.ShapeDtypeStruct((_N, _D), jnp.float32),
    scratch_shapes=[
        pltpu.VMEM((_N, _D), jnp.float32),
        pltpu.VMEM((2, _D), jnp.float32),
    ],
)


def _conv3_head_body(x_ref, a0_ref, a1_ref, w1_ref, b1_ref, g_ref, bt_ref,
                     w2_ref, b2_ref, h1_ref, h2_ref, l1w_ref, l1b_ref,
                     l2w_ref, l2b_ref, o1_ref, o2_ref, hpre_ref, st_ref):
    p = pl.program_id(0)
    i = pl.program_id(1)

    @pl.when(p == 0)
    def _():
        _phase0(i, x_ref, a0_ref, a1_ref, w1_ref, b1_ref, hpre_ref, st_ref)

    @pl.when(p == 1)
    def _():
        hn = _normalize(i, g_ref, bt_ref, hpre_ref, st_ref)
        h3 = jnp.maximum(
            jnp.dot(hn, w2_ref[...], preferred_element_type=jnp.float32)
            + b2_ref[...], 0.0)
        t = (jnp.dot(h1_ref[...], l1w_ref[0:_D, :],
                     preferred_element_type=jnp.float32)
             + jnp.dot(h2_ref[...], l1w_ref[_D:2 * _D, :],
                       preferred_element_type=jnp.float32)
             + jnp.dot(h3, l1w_ref[2 * _D:3 * _D, :],
                       preferred_element_type=jnp.float32)
             + l1b_ref[...])
        t = jnp.maximum(t, 0.0)
        z = (jnp.dot(t, l2w_ref[...], preferred_element_type=jnp.float32)
             + l2b_ref[...])
        o1_ref[...] = z
        m = jnp.max(z, axis=1, keepdims=True)
        e = jnp.exp(z - m)
        o2_ref[...] = e / jnp.sum(e, axis=1, keepdims=True)


_conv3_head_tc = pl.pallas_call(
    _conv3_head_body,
    grid=(2, _GRID),
    in_specs=[
        pl.BlockSpec((_BLK, _D), _blk_map),
        pl.BlockSpec((_BLK, _D), _blk_map),
        pl.BlockSpec((_BLK, _D), _blk_map),
        pl.BlockSpec((_D, _D), _full_map),
        pl.BlockSpec((1, _D), _full_map),
        pl.BlockSpec((1, _D), _full_map),
        pl.BlockSpec((1, _D), _full_map),
        pl.BlockSpec((_D, _D), _full_map),
        pl.BlockSpec((1, _D), _full_map),
        pl.BlockSpec((_BLK, _D), _blk_map2),
        pl.BlockSpec((_BLK, _D), _blk_map2),
        pl.BlockSpec((3 * _D, 3 * _D), _full_map),
        pl.BlockSpec((1, 3 * _D), _full_map),
        pl.BlockSpec((3 * _D, _D), _full_map),
        pl.BlockSpec((1, _D), _full_map),
    ],
    out_specs=[
        pl.BlockSpec((_BLK, _D), _blk_map2),
        pl.BlockSpec((_BLK, _D), _blk_map2),
    ],
    out_shape=[
        jax.ShapeDtypeStruct((_N, _D), jnp.float32),
        jax.ShapeDtypeStruct((_N, _D), jnp.float32),
    ],
    scratch_shapes=[
        pltpu.VMEM((_N, _D), jnp.float32),
        pltpu.VMEM((2, _D), jnp.float32),
    ],
)


def kernel(x, edge_index,
           c1_W1, c1_b1, c1_g, c1_bt, c1_W2, c1_b2,
           c2_W1, c2_b1, c2_g, c2_bt, c2_W2, c2_b2,
           c3_W1, c3_b1, c3_g, c3_bt, c3_W2, c3_b2,
           lin1_W, lin1_b, lin2_W, lin2_b):
    src3 = edge_index[0].reshape(_NW, _NCH, _CH)
    dst3 = edge_index[1].reshape(_NW, _NCH, _CH)
    zeros = jnp.zeros((_NP, _D), jnp.float32)

    def conv(xin, W1, b1, g, bt, W2, b2):
        agg = _sc_agg(xin, src3, dst3, zeros)
        return _conv_tc(xin, agg[:_N], agg[_NP:_NP + _N], W1,
                        b1.reshape(1, _D), g.reshape(1, _D),
                        bt.reshape(1, _D), W2, b2.reshape(1, _D))

    h1 = conv(x, c1_W1, c1_b1, c1_g, c1_bt, c1_W2, c1_b2)
    h2 = conv(h1, c2_W1, c2_b1, c2_g, c2_bt, c2_W2, c2_b2)
    agg3 = _sc_agg(h2, src3, dst3, zeros)
    return tuple(_conv3_head_tc(
        h2, agg3[:_N], agg3[_NP:_NP + _N], c3_W1, c3_b1.reshape(1, _D),
        c3_g.reshape(1, _D), c3_bt.reshape(1, _D), c3_W2,
        c3_b2.reshape(1, _D), h1, h2, lin1_W, lin1_b.reshape(1, 3 * _D),
        lin2_W, lin2_b.reshape(1, _D)))


# speedup vs baseline: 1.0315x; 1.0315x over previous
import functools

import jax
import jax.numpy as jnp
from jax import lax
from jax.experimental import pallas as pl
from jax.experimental.pallas import tpu as pltpu
from jax.experimental.pallas import tpu_sc as plsc

_N = 10000
_E = 320000
_D = 128

_NC = 2
_NS = 16
_NW = _NC * _NS
_EPW = _E // _NW
_CH = 80
_NCH = _EPW // _CH
_HCH = 64
_NP = 10240
_RPT = _NP // _NS

@functools.cache
def _make_sc_agg():
    mesh = plsc.VectorSubcoreMesh(core_axis_name="c", subcore_axis_name="s")

    @functools.partial(
        pl.kernel,
        out_type=jax.ShapeDtypeStruct((_NC * _NP, _D), jnp.float32),
        mesh=mesh,
        scratch_types=[
            pltpu.VMEM((_HCH, _CH), jnp.int32),
            pltpu.VMEM((_HCH, _CH), jnp.int32),
            pltpu.VMEM((_CH, _D), jnp.float32),
            pltpu.VMEM((_CH, _D), jnp.float32),
            pltpu.VMEM((_CH, _D), jnp.float32),
            pltpu.VMEM_SHARED((_NP, _D), jnp.float32),
            pltpu.SemaphoreType.DMA,
            pltpu.SemaphoreType.DMA,
            pltpu.SemaphoreType.DMA,
        ],
    )
    def sc_agg(x_hbm, src_hbm, dst_hbm, zeros_hbm, out_hbm,
               src_v, dst_v, rows_a, rows_b, rows_c, agg_sh,
               sem_a, sem_b, sem_c):
        cid = lax.axis_index("c")
        sid = lax.axis_index("s")
        wid = cid * _NS + sid
        row0 = sid * _RPT
        pltpu.sync_copy(zeros_hbm.at[pl.ds(row0, _RPT)],
                        agg_sh.at[pl.ds(row0, _RPT)])
        plsc.subcore_barrier()

        def gather(j, rv, sem):
            pltpu.async_copy(x_hbm.at[src_v.at[j]], rv, sem)

        def drain_scatter(j, rv, sem):
            pltpu.make_async_copy(x_hbm.at[src_v.at[j]], rv, sem).wait()
            pltpu.sync_copy(rv, agg_sh.at[dst_v.at[j]], add=True)

        def half(base, n):
            pltpu.sync_copy(src_hbm.at[wid, pl.ds(base, n)],
                            src_v.at[pl.ds(0, n)])
            pltpu.sync_copy(dst_hbm.at[wid, pl.ds(base, n)],
                            dst_v.at[pl.ds(0, n)])
            gather(0, rows_a, sem_a)
            gather(1, rows_b, sem_b)
            gather(2, rows_c, sem_c)
            ntr = (n - 4) // 3

            def body(i, carry):
                ja = 3 * i
                drain_scatter(ja, rows_a, sem_a)
                gather(ja + 3, rows_a, sem_a)
                drain_scatter(ja + 1, rows_b, sem_b)
                gather(ja + 4, rows_b, sem_b)
                drain_scatter(ja + 2, rows_c, sem_c)
                gather(ja + 5, rows_c, sem_c)
                return carry

            lax.fori_loop(0, ntr, body, 0)
            j0 = n - 4
            drain_scatter(j0, rows_a, sem_a)
            gather(n - 1, rows_a, sem_a)
            drain_scatter(j0 + 1, rows_b, sem_b)
            drain_scatter(j0 + 2, rows_c, sem_c)
            drain_scatter(n - 1, rows_a, sem_a)

        half(0, _HCH)
        half(_HCH, _NCH - _HCH)
        plsc.subcore_barrier()
        pltpu.sync_copy(agg_sh.at[pl.ds(row0, _RPT)],
                        out_hbm.at[pl.ds(cid * _NP + row0, _RPT)])

    return sc_agg


def _sc_agg(x, src3, dst3, zeros):
    return _make_sc_agg()(x, src3, dst3, zeros)


_BLK = 2000
_GRID = _N // _BLK


def _phase0(i, x_ref, a0_ref, a1_ref, w1_ref, b1_ref, hpre_ref, st_ref):
    s = x_ref[...] + a0_ref[...] + a1_ref[...]
    h = (jnp.dot(s, w1_ref[...], preferred_element_type=jnp.float32)
         + b1_ref[...])
    hpre_ref[pl.ds(i * _BLK, _BLK), :] = h

    @pl.when(i == 0)
    def _():
        st_ref[...] = jnp.zeros_like(st_ref)

    su = jnp.sum(h, axis=0, keepdims=True)
    sq = jnp.sum(h * h, axis=0, keepdims=True)
    st_ref[...] += jnp.concatenate([su, sq], axis=0)


def _normalize(i, g_ref, bt_ref, hpre_ref, st_ref):
    mean = st_ref[0:1, :] * (1.0 / _N)
    var = st_ref[1:2, :] * (1.0 / _N) - mean * mean
    scale = g_ref[...] * lax.rsqrt(var + 128.0)
    h = hpre_ref[pl.ds(i * _BLK, _BLK), :]
    return jnp.maximum((h - mean) * scale + bt_ref[...], 0.0)


def _conv_body(x_ref, a0_ref, a1_ref, w1_ref, b1_ref, g_ref, bt_ref,
               w2_ref, b2_ref, o_ref, hpre_ref, st_ref):
    p = pl.program_id(0)
    i = pl.program_id(1)

    @pl.when(p == 0)
    def _():
        _phase0(i, x_ref, a0_ref, a1_ref, w1_ref, b1_ref, hpre_ref, st_ref)

    @pl.when(p == 1)
    def _():
        hn = _normalize(i, g_ref, bt_ref, hpre_ref, st_ref)
        o_ref[...] = jnp.maximum(
            jnp.dot(hn, w2_ref[...], preferred_element_type=jnp.float32)
            + b2_ref[...], 0.0)


def _blk_map(p, i):
    return (i * (1 - p), 0)


def _blk_map2(p, i):
    return (i, 0)


def _full_map(p, i):
    return (0, 0)


_conv_tc = pl.pallas_call(
    _conv_body,
    grid=(2, _GRID),
    in_specs=[
        pl.BlockSpec((_BLK, _D), _blk_map),
        pl.BlockSpec((_BLK, _D), _blk_map),
        pl.BlockSpec((_BLK, _D), _blk_map),
        pl.BlockSpec((_D, _D), _full_map),
        pl.BlockSpec((1, _D), _full_map),
        pl.BlockSpec((1, _D), _full_map),
        pl.BlockSpec((1, _D), _full_map),
        pl.BlockSpec((_D, _D), _full_map),
        pl.BlockSpec((1, _D), _full_map),
    ],
    out_specs=pl.BlockSpec((_BLK, _D), _blk_map2),
    out_shape=jax.ShapeDtypeStruct((_N, _D), jnp.float32),
    scratch_shapes=[
        pltpu.VMEM((_N, _D), jnp.float32),
        pltpu.VMEM((2, _D), jnp.float32),
    ],
)


def _conv3_head_body(x_ref, a0_ref, a1_ref, w1_ref, b1_ref, g_ref, bt_ref,
                     w2_ref, b2_ref, h1_ref, h2_ref, l1w_ref, l1b_ref,
                     l2w_ref, l2b_ref, o1_ref, o2_ref, hpre_ref, st_ref):
    p = pl.program_id(0)
    i = pl.program_id(1)

    @pl.when(p == 0)
    def _():
        _phase0(i, x_ref, a0_ref, a1_ref, w1_ref, b1_ref, hpre_ref, st_ref)

    @pl.when(p == 1)
    def _():
        hn = _normalize(i, g_ref, bt_ref, hpre_ref, st_ref)
        h3 = jnp.maximum(
            jnp.dot(hn, w2_ref[...], preferred_element_type=jnp.float32)
            + b2_ref[...], 0.0)
        t = (jnp.dot(h1_ref[...], l1w_ref[0:_D, :],
                     preferred_element_type=jnp.float32)
             + jnp.dot(h2_ref[...], l1w_ref[_D:2 * _D, :],
                       preferred_element_type=jnp.float32)
             + jnp.dot(h3, l1w_ref[2 * _D:3 * _D, :],
                       preferred_element_type=jnp.float32)
             + l1b_ref[...])
        t = jnp.maximum(t, 0.0)
        z = (jnp.dot(t, l2w_ref[...], preferred_element_type=jnp.float32)
             + l2b_ref[...])
        o1_ref[...] = z
        m = jnp.max(z, axis=1, keepdims=True)
        e = jnp.exp(z - m)
        o2_ref[...] = e / jnp.sum(e, axis=1, keepdims=True)


_conv3_head_tc = pl.pallas_call(
    _conv3_head_body,
    grid=(2, _GRID),
    in_specs=[
        pl.BlockSpec((_BLK, _D), _blk_map),
        pl.BlockSpec((_BLK, _D), _blk_map),
        pl.BlockSpec((_BLK, _D), _blk_map),
        pl.BlockSpec((_D, _D), _full_map),
        pl.BlockSpec((1, _D), _full_map),
        pl.BlockSpec((1, _D), _full_map),
        pl.BlockSpec((1, _D), _full_map),
        pl.BlockSpec((_D, _D), _full_map),
        pl.BlockSpec((1, _D), _full_map),
        pl.BlockSpec((_BLK, _D), _blk_map2),
        pl.BlockSpec((_BLK, _D), _blk_map2),
        pl.BlockSpec((3 * _D, 3 * _D), _full_map),
        pl.BlockSpec((1, 3 * _D), _full_map),
        pl.BlockSpec((3 * _D, _D), _full_map),
        pl.BlockSpec((1, _D), _full_map),
    ],
    out_specs=[
        pl.BlockSpec((_BLK, _D), _blk_map2),
        pl.BlockSpec((_BLK, _D), _blk_map2),
    ],
    out_shape=[
        jax.ShapeDtypeStruct((_N, _D), jnp.float32),
        jax.ShapeDtypeStruct((_N, _D), jnp.float32),
    ],
    scratch_shapes=[
        pltpu.VMEM((_N, _D), jnp.float32),
        pltpu.VMEM((2, _D), jnp.float32),
    ],
)


def kernel(x, edge_index,
           c1_W1, c1_b1, c1_g, c1_bt, c1_W2, c1_b2,
           c2_W1, c2_b1, c2_g, c2_bt, c2_W2, c2_b2,
           c3_W1, c3_b1, c3_g, c3_bt, c3_W2, c3_b2,
           lin1_W, lin1_b, lin2_W, lin2_b):
    src3 = edge_index[0].reshape(_NW, _NCH, _CH)
    dst3 = edge_index[1].reshape(_NW, _NCH, _CH)
    zeros = jnp.zeros((_NP, _D), jnp.float32)

    def conv(xin, W1, b1, g, bt, W2, b2):
        agg = _sc_agg(xin, src3, dst3, zeros)
        return _conv_tc(xin, agg[:_N], agg[_NP:_NP + _N], W1,
                        b1.reshape(1, _D), g.reshape(1, _D),
                        bt.reshape(1, _D), W2, b2.reshape(1, _D))

    h1 = conv(x, c1_W1, c1_b1, c1_g, c1_bt, c1_W2, c1_b2)
    h2 = conv(h1, c2_W1, c2_b1, c2_g, c2_bt, c2_W2, c2_b2)
    agg3 = _sc_agg(h2, src3, dst3, zeros)
    return tuple(_conv3_head_tc(
        h2, agg3[:_N], agg3[_NP:_NP + _N], c3_W1, c3_b1.reshape(1, _D),
        c3_g.reshape(1, _D), c3_bt.reshape(1, _D), c3_W2,
        c3_b2.reshape(1, _D), h1, h2, lin1_W, lin1_b.reshape(1, 3 * _D),
        lin2_W, lin2_b.reshape(1, _D)))
